# Initial kernel scaffold; baseline (speedup 1.0000x reference)
import jax
import jax.numpy as jnp
from jax.experimental import pallas as pl


def kernel(x, edge_attr, params, edge_index, batch):
    raise NotImplementedError("write your pallas kernel here")


# speedup vs baseline: 14.8954x; 14.8954x over previous
import functools

import jax
import jax.numpy as jnp
import numpy as np
from jax import lax
from jax.experimental import pallas as pl
from jax.experimental.pallas import tpu as pltpu
from jax.experimental.pallas import tpu_sc as plsc

N = 10000
NP = 10240
NT_TILES = NP // 128
E = 320000
DF = 128
DE = 16
H = 128
NH = 8
HD = 16
NL = 4
NTASK = 5
NG = 64
EC = 1280
NCHUNK = E // EC
EHALF = E // 2
NCHUNK_H = EHALF // EC
ETILE = 512
NE_TILES = E // ETILE

_SC_PARAMS = pltpu.CompilerParams(needs_layout_passes=False)


def _eye(n):
    a = lax.broadcasted_iota(jnp.int32, (n, n), 0)
    b = lax.broadcasted_iota(jnp.int32, (n, n), 1)
    return (a == b).astype(jnp.float32)


def _tr(x):
    return lax.dot_general(x, _eye(x.shape[0]), (((0,), (0,)), ((), ())),
                           preferred_element_type=jnp.float32)


def _dot(a, b):
    return lax.dot_general(a, b, (((1,), (0,)), ((), ())),
                           preferred_element_type=jnp.float32)


def _col_mask(rows, r):
    c = lax.broadcasted_iota(jnp.int32, (rows, 128), 1) + r * 128
    return c < N


def _prep_kernel(attr_ref, m1t_ref, call_ref, attrT_ref, aeT_ref, aemax_ref):
    e = pl.program_id(0)
    blk = attr_ref[...]
    t = lax.dot_general(blk, _eye(ETILE), (((0,), (0,)), ((), ())),
                        preferred_element_type=jnp.float32)
    attrT_ref[...] = t
    ae = _dot(m1t_ref[...], t) + call_ref[...][:, 0:1]
    aeT_ref[...] = ae
    m = jnp.max(ae.reshape(32, ETILE // 128, 128), axis=1)

    @pl.when(e == 0)
    def _():
        aemax_ref[...] = jnp.full((32, 128), -3e38, jnp.float32)

    aemax_ref[...] = jnp.maximum(aemax_ref[...], m)


def _run_prep(edge_attr, M1_T_all, c_all2d):
    return pl.pallas_call(
        _prep_kernel,
        grid=(NE_TILES,),
        in_specs=[
            pl.BlockSpec((ETILE, DE), lambda e: (e, 0)),
            pl.BlockSpec((32, DE), lambda e: (0, 0)),
            pl.BlockSpec((32, 128), lambda e: (0, 0)),
        ],
        out_specs=[
            pl.BlockSpec((DE, ETILE), lambda e: (0, e)),
            pl.BlockSpec((32, ETILE), lambda e: (0, e)),
            pl.BlockSpec((32, 128), lambda e: (0, 0)),
        ],
        out_shape=[
            jax.ShapeDtypeStruct((DE, E), jnp.float32),
            jax.ShapeDtypeStruct((32, E), jnp.float32),
            jax.ShapeDtypeStruct((32, 128), jnp.float32),
        ],
    )(edge_attr, M1_T_all, c_all2d)


def _sc_avg_build():
    mesh = plsc.VectorSubcoreMesh(core_axis_name="c", subcore_axis_name="s")

    @functools.partial(
        pl.kernel, mesh=mesh, compiler_params=_SC_PARAMS,
        out_type=(jax.ShapeDtypeStruct((32 * NP,), jnp.float32),
                  jax.ShapeDtypeStruct((2 * NP,), jnp.float32)),
        scratch_types=[
            pltpu.VMEM((NP,), jnp.float32),
            pltpu.VMEM((NP,), jnp.float32),
            pltpu.VMEM((EC,), jnp.int32),
            pltpu.VMEM((EC,), jnp.float32),
        ],
    )
    def k(attrT_hbm, dst_hbm, s_hbm, deg_hbm, acc_v, deg_v, dst_v, val_v):
        wid = lax.axis_index("s") * 2 + lax.axis_index("c")
        f = wid % 16
        h2 = wid // 16
        ebase = h2 * EHALF
        zero = jnp.zeros((16,), jnp.float32)
        ones = jnp.ones((16,), jnp.float32)

        def zinit(i, _):
            acc_v[pl.ds(i * 16, 16)] = zero
            deg_v[pl.ds(i * 16, 16)] = zero
            return 0

        lax.fori_loop(0, NP // 16, zinit, 0)

        def chunk(ci, _):
            base = ebase + ci * EC
            pltpu.sync_copy(dst_hbm.at[pl.ds(base, EC)], dst_v)
            pltpu.sync_copy(attrT_hbm.at[pl.ds(f * E + base, EC)], val_v)

            def inner(i, _):
                d16 = dst_v[pl.ds(i * 16, 16)]
                v16 = val_v[pl.ds(i * 16, 16)]
                plsc.addupdate_scatter(acc_v, [d16], v16)

                @pl.when(f == 0)
                def _():
                    plsc.addupdate_scatter(deg_v, [d16], ones)

                return 0

            lax.fori_loop(0, EC // 16, inner, 0, unroll=4)
            return 0

        lax.fori_loop(0, NCHUNK_H, chunk, 0)
        pltpu.sync_copy(acc_v, s_hbm.at[pl.ds((h2 * 16 + f) * NP, NP)])

        @pl.when(f == 0)
        def _():
            pltpu.sync_copy(deg_v, deg_hbm.at[pl.ds(h2 * NP, NP)])

    return k


def _sc_gat_build(layer):
    mesh = plsc.VectorSubcoreMesh(core_axis_name="c", subcore_axis_name="s")

    @functools.partial(
        pl.kernel, mesh=mesh, compiler_params=_SC_PARAMS,
        out_type=(jax.ShapeDtypeStruct((128 * NP,), jnp.float32),
                  jax.ShapeDtypeStruct((8 * NP,), jnp.float32)),
        scratch_types=[
            pltpu.VMEM((4 * NP,), jnp.float32),
            pltpu.VMEM((4 * NP,), jnp.float32),
            pltpu.VMEM((NP,), jnp.float32),
            pltpu.VMEM((NP,), jnp.float32),
            pltpu.VMEM((NP,), jnp.float32),
            pltpu.VMEM((EC,), jnp.int32),
            pltpu.VMEM((EC,), jnp.int32),
            pltpu.VMEM((EC,), jnp.float32),
            pltpu.VMEM((EC,), jnp.float32),
            pltpu.VMEM((128,), jnp.float32),
            pltpu.VMEM((128,), jnp.float32),
            pltpu.VMEM((128,), jnp.float32),
            pltpu.VMEM((128,), jnp.float32),
        ],
    )
    def k(xp_hbm, asrc_hbm, adst_hbm, ae_hbm, src_hbm, dst_hbm,
          cs_hbm, cd_hbm, cf_hbm, ce_hbm,
          raw_hbm, den_hbm,
          xp_v, acc_v, asrc_v, adst_v, den_v, src_v, dst_v, ae_v, ex_v,
          cs_v, cd_v, cf_v, ce_v):
        wid = lax.axis_index("s") * 2 + lax.axis_index("c")
        h = wid // 4
        q = wid % 4
        f0 = wid * 4
        pltpu.sync_copy(xp_hbm.at[pl.ds(f0 * NP, 4 * NP)], xp_v)
        pltpu.sync_copy(asrc_hbm.at[pl.ds(h * NP, NP)], asrc_v)
        pltpu.sync_copy(adst_hbm.at[pl.ds(h * NP, NP)], adst_v)
        pltpu.sync_copy(cs_hbm.at[pl.ds(h * 128, 128)], cs_v)
        pltpu.sync_copy(cd_hbm.at[pl.ds(h * 128, 128)], cd_v)
        pltpu.sync_copy(cf_hbm.at[pl.ds(h * 128, 128)], cf_v)
        pltpu.sync_copy(ce_hbm.at[pl.ds(h * 128, 128)], ce_v)

        def vmax(v):
            def mstep(i, m):
                return jnp.maximum(m, v[pl.ds(i * 16, 16)])
            m = lax.fori_loop(0, 8, mstep, jnp.full((16,), -3e38, jnp.float32))
            return jnp.max(m)

        c = vmax(cs_v) + vmax(cd_v) + jnp.maximum(vmax(cf_v), vmax(ce_v))
        cvec = jnp.broadcast_to(c, (16,))

        zero = jnp.zeros((16,), jnp.float32)

        def zinit(i, _):
            acc_v[pl.ds(i * 16, 16)] = zero
            return 0

        lax.fori_loop(0, 4 * NP // 16, zinit, 0)

        def zden(i, _):
            den_v[pl.ds(i * 16, 16)] = zero
            return 0

        lax.fori_loop(0, NP // 16, zden, 0)

        iota = lax.iota(jnp.int32, 16)
        lane4 = iota & 3
        grp4 = iota >> 2

        def chunk(ci, _):
            base = ci * EC
            pltpu.sync_copy(src_hbm.at[pl.ds(base, EC)], src_v)
            pltpu.sync_copy(dst_hbm.at[pl.ds(base, EC)], dst_v)
            pltpu.sync_copy(ae_hbm.at[pl.ds((8 * layer + h) * E + base, EC)], ae_v)

            def exb(i, _):
                s16 = src_v[pl.ds(i * 16, 16)]
                d16 = dst_v[pl.ds(i * 16, 16)]
                a = (plsc.load_gather(asrc_v, [s16])
                     + plsc.load_gather(adst_v, [d16])
                     + ae_v[pl.ds(i * 16, 16)])
                a = jnp.where(a > 0, a, 0.2 * a)
                ex = jnp.exp(a - cvec)
                ex_v[pl.ds(i * 16, 16)] = ex

                @pl.when(q == 0)
                def _():
                    plsc.addupdate_scatter(den_v, [d16], ex)

                return 0

            lax.fori_loop(0, EC // 16, exb, 0, unroll=4)

            def msg(j, _):
                eidx = j * 4 + grp4
                sx = plsc.load_gather(src_v, [eidx])
                dx = plsc.load_gather(dst_v, [eidx])
                exx = plsc.load_gather(ex_v, [eidx])
                val = plsc.load_gather(xp_v, [lane4 * NP + sx])
                plsc.addupdate_scatter(acc_v, [lane4 * NP + dx], val * exx)
                return 0

            lax.fori_loop(0, EC // 4, msg, 0, unroll=4)
            return 0

        lax.fori_loop(0, NCHUNK, chunk, 0)
        pltpu.sync_copy(acc_v, raw_hbm.at[pl.ds(f0 * NP, 4 * NP)])

        @pl.when(q == 0)
        def _():
            pltpu.sync_copy(den_v, den_hbm.at[pl.ds(h * NP, NP)])

    return k


def _a_phase(r, hT, avgT, hd8, wT, asT, adT, m1T, ccol,
             xp_ref, asrc_ref, adst_ref, aself_ref, cs_ref, cd_ref, cf_ref):
    xp = _dot(wT, hT)
    asrc = _dot(asT, hT)
    adst = _dot(adT, hT)
    aself = _dot(m1T, avgT) + ccol * hd8
    xp_ref[...] = xp
    asrc_ref[...] = asrc
    adst_ref[...] = adst
    aself_ref[...] = aself

    @pl.when(r == 0)
    def _():
        neg = jnp.full((8, 128), -3e38, jnp.float32)
        cs_ref[...] = neg
        cd_ref[...] = neg
        cf_ref[...] = neg

    cs_ref[...] = jnp.maximum(cs_ref[...], asrc)
    cd_ref[...] = jnp.maximum(cd_ref[...], adst)
    cf_ref[...] = jnp.maximum(cf_ref[...], aself)


_A_OUT_SHAPES = [
    jax.ShapeDtypeStruct((H, NP), jnp.float32),
    jax.ShapeDtypeStruct((8, NP), jnp.float32),
    jax.ShapeDtypeStruct((8, NP), jnp.float32),
    jax.ShapeDtypeStruct((8, NP), jnp.float32),
    jax.ShapeDtypeStruct((8, 128), jnp.float32),
    jax.ShapeDtypeStruct((8, 128), jnp.float32),
    jax.ShapeDtypeStruct((8, 128), jnp.float32),
]

_A_OUT_SPECS = [
    pl.BlockSpec((H, 128), lambda r: (0, r)),
    pl.BlockSpec((8, 128), lambda r: (0, r)),
    pl.BlockSpec((8, 128), lambda r: (0, r)),
    pl.BlockSpec((8, 128), lambda r: (0, r)),
    pl.BlockSpec((8, 128), lambda r: (0, 0)),
    pl.BlockSpec((8, 128), lambda r: (0, 0)),
    pl.BlockSpec((8, 128), lambda r: (0, 0)),
]

_W_SPEC = pl.BlockSpec((H, H), lambda r: (0, 0))
_W8_SPEC = pl.BlockSpec((8, H), lambda r: (0, 0))
_C_SPEC = pl.BlockSpec((8, 128), lambda r: (0, 0))
_TILE_SPEC = pl.BlockSpec((H, 128), lambda r: (0, r))
_ROW8_SPEC = pl.BlockSpec((8, 128), lambda r: (0, r))


def _a0_kernel(x_ref, s_ref, deg_ref, watom_ref, batom_ref,
               wT_ref, asT_ref, adT_ref, m1T_ref, ccol_ref,
               hT_ref, avgT_ref, hd8_ref,
               xp_ref, asrc_ref, adst_ref, aself_ref, cs_ref, cd_ref, cf_ref):
    r = pl.program_id(0)
    h0 = _dot(x_ref[...], watom_ref[...]) + batom_ref[...]
    hT = _tr(h0)
    hT = jnp.where(_col_mask(H, r), hT, 0.0)
    hT_ref[...] = hT
    s = s_ref[...]
    d2 = deg_ref[...]
    d = d2[0:1] + d2[1:2]
    avgT = (s[0] + s[1]) / jnp.maximum(d, 1.0)
    hd8 = jnp.broadcast_to((d > 0).astype(jnp.float32), (8, 128))
    avgT_ref[...] = avgT
    hd8_ref[...] = hd8
    _a_phase(r, hT, avgT, hd8, wT_ref[...], asT_ref[...], adT_ref[...],
             m1T_ref[...], ccol_ref[...][:, 0:1],
             xp_ref, asrc_ref, adst_ref, aself_ref, cs_ref, cd_ref, cf_ref)


def _run_a0(x_p, S3, degp, p, wd):
    return pl.pallas_call(
        _a0_kernel,
        grid=(NT_TILES,),
        in_specs=[
            pl.BlockSpec((128, DF), lambda r: (r, 0)),
            pl.BlockSpec((2, 16, 128), lambda r: (0, 0, r)),
            pl.BlockSpec((2, 128), lambda r: (0, r)),
            _W_SPEC,
            _W_SPEC,
            _W_SPEC, _W8_SPEC, _W8_SPEC,
            pl.BlockSpec((8, DE), lambda r: (0, 0)),
            _C_SPEC,
        ],
        out_specs=[_TILE_SPEC, pl.BlockSpec((16, 128), lambda r: (0, r)),
                   _ROW8_SPEC] + _A_OUT_SPECS,
        out_shape=[
            jax.ShapeDtypeStruct((H, NP), jnp.float32),
            jax.ShapeDtypeStruct((16, NP), jnp.float32),
            jax.ShapeDtypeStruct((8, NP), jnp.float32),
        ] + _A_OUT_SHAPES,
    )(x_p, S3, degp, p['W_atom'], wd['batom2d'],
      wd['wT'][0], wd['asT'][0], wd['adT'][0], wd['m1T'][0], wd['ccol'][0])


def _b1_kernel(raw_ref, den_ref, xp_ref, asrc_ref, adst_ref, aself_ref,
               cs_ref, cd_ref, cf_ref, ce_ref, bias_ref,
               o_ref, sacc_ref, ssacc_ref):
    r = pl.program_id(0)
    ch = (jnp.max(cs_ref[...], axis=1, keepdims=True)
          + jnp.max(cd_ref[...], axis=1, keepdims=True)
          + jnp.maximum(jnp.max(cf_ref[...], axis=1, keepdims=True),
                        jnp.max(ce_ref[...], axis=1, keepdims=True)))
    al = asrc_ref[...] + adst_ref[...] + aself_ref[...]
    al = jnp.where(al > 0, al, 0.2 * al)
    exs = jnp.exp(al - ch)
    dent = den_ref[...] + exs
    e128 = jnp.reshape(jnp.broadcast_to(exs[:, None, :], (8, 16, 128)), (128, 128))
    d128 = jnp.reshape(jnp.broadcast_to(dent[:, None, :], (8, 16, 128)), (128, 128))
    o = (raw_ref[...] + e128 * xp_ref[...]) / d128 + bias_ref[...]
    om = jnp.where(_col_mask(H, r), o, 0.0)
    o_ref[...] = om

    @pl.when(r == 0)
    def _():
        z = jnp.zeros((H, 128), jnp.float32)
        sacc_ref[...] = z
        ssacc_ref[...] = z

    sacc_ref[...] = sacc_ref[...] + om
    ssacc_ref[...] = ssacc_ref[...] + om * om


def _run_b1(rawT, denT, xpT, asrcT, adstT, aselfT, cs, cd, cf, ce, bias2d):
    return pl.pallas_call(
        _b1_kernel,
        grid=(NT_TILES,),
        in_specs=[_TILE_SPEC, _ROW8_SPEC, _TILE_SPEC, _ROW8_SPEC, _ROW8_SPEC,
                  _ROW8_SPEC, _C_SPEC, _C_SPEC, _C_SPEC, _C_SPEC, _W_SPEC],
        out_specs=[_TILE_SPEC, pl.BlockSpec((H, 128), lambda r: (0, 0)),
                   pl.BlockSpec((H, 128), lambda r: (0, 0))],
        out_shape=[jax.ShapeDtypeStruct((H, NP), jnp.float32),
                   jax.ShapeDtypeStruct((H, 128), jnp.float32),
                   jax.ShapeDtypeStruct((H, 128), jnp.float32)],
    )(rawT, denT, xpT, asrcT, adstT, aselfT, cs, cd, cf, ce, bias2d)


def _make_b2_kernel(layer, last):
    def body(o_ref, sacc_ref, ssacc_ref, g_ref, b_ref, hres_ref,
             avgT_ref, hd8_ref, wT_ref, asT_ref, adT_ref, m1T_ref, ccol_ref,
             hT_ref, *a_refs):
        r = pl.program_id(0)
        srow = jnp.sum(sacc_ref[...], axis=1, keepdims=True)
        ssrow = jnp.sum(ssacc_ref[...], axis=1, keepdims=True)
        mu = srow / float(N)
        var = ssrow / float(N) - mu * mu
        inv = lax.rsqrt(var + 1e-5)
        hn = (o_ref[...] - mu) * inv * g_ref[...] + b_ref[...]
        hn = jnp.maximum(hn, 0.0)
        if layer > 0:
            hn = hn + hres_ref[...]
        hn = jnp.where(_col_mask(H, r), hn, 0.0)
        hT_ref[...] = hn
        if not last:
            _a_phase(r, hn, avgT_ref[...], hd8_ref[...], wT_ref[...],
                     asT_ref[...], adT_ref[...], m1T_ref[...],
                     ccol_ref[...][:, 0:1], *a_refs)
    return body


def _run_b2(layer, oT, sacc, ssacc, g2d, b2d, hresT, avgT, hd8T, wd):
    last = layer == NL - 1
    nxt = layer + 1
    out_specs = [_TILE_SPEC]
    out_shape = [jax.ShapeDtypeStruct((H, NP), jnp.float32)]
    if not last:
        out_specs += _A_OUT_SPECS
        out_shape += _A_OUT_SHAPES
    return pl.pallas_call(
        _make_b2_kernel(layer, last),
        grid=(NT_TILES,),
        in_specs=[_TILE_SPEC,
                  pl.BlockSpec((H, 128), lambda r: (0, 0)),
                  pl.BlockSpec((H, 128), lambda r: (0, 0)),
                  _W_SPEC, _W_SPEC, _TILE_SPEC,
                  pl.BlockSpec((16, 128), lambda r: (0, r)),
                  _ROW8_SPEC,
                  _W_SPEC, _W8_SPEC, _W8_SPEC,
                  pl.BlockSpec((8, DE), lambda r: (0, 0)),
                  _C_SPEC],
        out_specs=out_specs,
        out_shape=out_shape,
    )(oT, sacc, ssacc, g2d, b2d, hresT, avgT, hd8T,
      wd['wT'][nxt % NL], wd['asT'][nxt % NL], wd['adT'][nxt % NL],
      wd['m1T'][nxt % NL], wd['ccol'][nxt % NL])


def _qkv_kernel(hT_ref, wqT_ref, wkT_ref, wvT_ref, bq_ref, bk_ref, bv_ref,
                q_ref, k_ref, v_ref):
    hnat = _tr(hT_ref[...])
    q_ref[...] = _dot(hnat, wqT_ref[...]) + bq_ref[...]
    k_ref[...] = _dot(hnat, wkT_ref[...]) + bk_ref[...]
    v_ref[...] = _dot(hnat, wvT_ref[...]) + bv_ref[...]


def _run_qkv(hT, wqT, wkT, wvT, bq2d, bk2d, bv2d):
    spec_nat = pl.BlockSpec((128, H), lambda r: (r, 0))
    return pl.pallas_call(
        _qkv_kernel,
        grid=(NT_TILES,),
        in_specs=[_TILE_SPEC] + [_W_SPEC] * 6,
        out_specs=[spec_nat] * 3,
        out_shape=[jax.ShapeDtypeStruct((NP, H), jnp.float32)] * 3,
    )(hT, wqT, wkT, wvT, bq2d, bk2d, bv2d)


def _mha_kernel(q_ref, k_ref, v_ref, brow_ref, btile_ref, woT_ref, bo_ref,
                xg_ref, cnt_ref):
    r = pl.program_id(0)
    btile = btile_ref[...]
    brT = lax.dot_general(_eye(128), btile, (((1,), (1,)), ((), ())),
                          preferred_element_type=jnp.float32)
    brow = brow_ref[...]
    mask = brT == brow
    q = q_ref[...]
    outs = []
    for hh in range(NH):
        sl = slice(hh * HD, (hh + 1) * HD)
        sc = lax.dot_general(q[:, sl], k_ref[:, sl],
                             (((1,), (1,)), ((), ())),
                             preferred_element_type=jnp.float32) * 0.25
        sc = jnp.where(mask, sc, -1e30)
        mx = jnp.max(sc, axis=1, keepdims=True)
        pexp = jnp.exp(sc - mx)
        den = jnp.sum(pexp, axis=1, keepdims=True)
        at = pexp / den
        outs.append(_dot(at, v_ref[:, sl]))
    o = jnp.concatenate(outs, axis=1)
    att = _dot(o, woT_ref[...]) + bo_ref[...]
    giota = lax.broadcasted_iota(jnp.int32, (NG, 128), 0).astype(jnp.float32)
    oneh = (giota == btile).astype(jnp.float32)
    xg = _dot(oneh, att)
    cnt = _dot(oneh, jnp.ones((128, 128), jnp.float32))

    @pl.when(r == 0)
    def _():
        z = jnp.zeros((NG, 128), jnp.float32)
        xg_ref[...] = z
        cnt_ref[...] = z

    xg_ref[...] = xg_ref[...] + xg
    cnt_ref[...] = cnt_ref[...] + cnt


def _run_mha(q, k, v, brow, woT, bo2d):
    return pl.pallas_call(
        _mha_kernel,
        grid=(NT_TILES,),
        in_specs=[
            pl.BlockSpec((128, H), lambda r: (r, 0)),
            pl.BlockSpec((NP, H), lambda r: (0, 0)),
            pl.BlockSpec((NP, H), lambda r: (0, 0)),
            pl.BlockSpec((1, NP), lambda r: (0, 0)),
            pl.BlockSpec((1, 128), lambda r: (0, r)),
            _W_SPEC, _W_SPEC,
        ],
        out_specs=[pl.BlockSpec((NG, 128), lambda r: (0, 0)),
                   pl.BlockSpec((NG, 128), lambda r: (0, 0))],
        out_shape=[jax.ShapeDtypeStruct((NG, 128), jnp.float32),
                   jax.ShapeDtypeStruct((NG, 128), jnp.float32)],
    )(q, k, v, brow, brow, woT, bo2d)


def _mlp_kernel(xg_ref, cnt_ref, w1_ref, b1_ref, w2_ref, b2_ref,
                wc1_ref, bc1_ref, wc2_ref, bc2_ref, out_ref):
    xg = xg_ref[...] / cnt_ref[...]
    s = jnp.maximum(_dot(xg, w1_ref[...]) + b1_ref[...], 0.0)
    s = jnp.maximum(_dot(s, w2_ref[...]) + b2_ref[...], 0.0)
    hh = jnp.maximum(_dot(s, wc1_ref[...]) + bc1_ref[...], 0.0)
    out_ref[...] = _dot(hh, wc2_ref[...]) + bc2_ref[...]


def _run_mlp(xg_sum, cnt, p, wd):
    return pl.pallas_call(
        _mlp_kernel,
        out_shape=jax.ShapeDtypeStruct((NG, NTASK), jnp.float32),
    )(xg_sum, cnt, p['sh_W1'], wd['shb1'], p['sh_W2'], wd['shb2'],
      wd['wc1'], wd['bc1'], wd['wc2'], wd['bc2'])


def kernel(x, edge_attr, params, edge_index, batch):
    p = params
    src = edge_index[0].astype(jnp.int32)
    dst = edge_index[1].astype(jnp.int32)

    asT, adT, m1T, ccol, wT = [], [], [], [], []
    M1_rows, c_list = [], []
    for i in range(NL):
        W = p[f'gat{i}_W']
        a_s = (W.reshape(H, NH, HD) * p[f'gat{i}_att_src'][0][None]).sum(-1)
        a_d = (W.reshape(H, NH, HD) * p[f'gat{i}_att_dst'][0][None]).sum(-1)
        Ep = (p[f'gat{i}_W_e'].reshape(H, NH, HD)
              * p[f'gat{i}_att_edge'][0][None]).sum(-1)
        M1 = p['W_edge_enc'] @ Ep
        cv = p['b_edge_enc'] @ Ep
        asT.append(a_s.T)
        adT.append(a_d.T)
        m1T.append(M1.T)
        ccol.append(jnp.broadcast_to(cv[:, None], (8, 128)))
        wT.append(W.T)
        M1_rows.append(M1.T)
        c_list.append(cv)
    wd = {
        'wT': wT, 'asT': asT, 'adT': adT, 'm1T': m1T, 'ccol': ccol,
        'batom2d': jnp.broadcast_to(p['b_atom'][None, :], (H, H)),
    }
    M1_T_all = jnp.concatenate(M1_rows, axis=0)
    c_all2d = jnp.broadcast_to(jnp.concatenate(c_list)[:, None], (32, 128))

    x_p = jnp.pad(x, ((0, NP - N), (0, 0)))
    batch_p = jnp.pad(batch.astype(jnp.int32), (0, NP - N),
                      constant_values=127)
    brow = batch_p.astype(jnp.float32).reshape(1, NP)

    attrT, aeT, aemax = _run_prep(edge_attr, M1_T_all, c_all2d)

    S_f, deg_f = _sc_avg_build()(attrT.reshape(-1), dst)
    S3 = S_f.reshape(2, 16, NP)
    degp = deg_f.reshape(2, NP)

    (hT, avgT, hd8T, xpT, asrcT, adstT, aselfT, cs, cd, cf) = _run_a0(
        x_p, S3, degp, p, wd)

    ae_flat = aeT.reshape(-1)
    for i in range(NL):
        ce = aemax[8 * i:8 * (i + 1)]
        raw_f, den_f = _sc_gat_build(i)(
            xpT.reshape(-1), asrcT.reshape(-1), adstT.reshape(-1), ae_flat,
            src, dst, cs.reshape(-1), cd.reshape(-1), cf.reshape(-1),
            ce.reshape(-1))
        rawT = raw_f.reshape(H, NP)
        denT = den_f.reshape(8, NP)
        bias2d = jnp.broadcast_to(p[f'gat{i}_bias'][:, None], (H, H))
        oT, sacc, ssacc = _run_b1(rawT, denT, xpT, asrcT, adstT, aselfT,
                                  cs, cd, cf, ce, bias2d)
        g2d = jnp.broadcast_to(p[f'bn{i}_g'][:, None], (H, H))
        b2d = jnp.broadcast_to(p[f'bn{i}_b'][:, None], (H, H))
        res = _run_b2(i, oT, sacc, ssacc, g2d, b2d, hT, avgT, hd8T, wd)
        if i < NL - 1:
            hT, xpT, asrcT, adstT, aselfT, cs, cd, cf = res
        else:
            hT = res[0]

    Wq, Wk, Wv = jnp.split(p['mha_in_w'], 3, axis=0)
    bq, bk, bv = jnp.split(p['mha_in_b'], 3)
    q, k, v = _run_qkv(hT, Wq.T, Wk.T, Wv.T,
                       jnp.broadcast_to(bq[None, :], (H, H)),
                       jnp.broadcast_to(bk[None, :], (H, H)),
                       jnp.broadcast_to(bv[None, :], (H, H)))
    xg_sum, cnt = _run_mha(q, k, v, brow, p['mha_out_w'].T,
                           jnp.broadcast_to(p['mha_out_b'][None, :], (H, H)))

    wc1 = jnp.concatenate([p[f'head{t}_W1'] for t in range(NTASK)], axis=1)
    bc1 = jnp.concatenate([p[f'head{t}_b1'] for t in range(NTASK)])[None, :]
    blocks = []
    for t in range(NTASK):
        col = jnp.zeros((H // 4, NTASK), jnp.float32)
        col = col.at[:, t].set(p[f'head{t}_W2'][:, 0])
        blocks.append(col)
    wc2 = jnp.concatenate(blocks, axis=0)
    bc2 = jnp.concatenate([p[f'head{t}_b2'] for t in range(NTASK)])[None, :]
    wd['shb1'] = p['sh_b1'][None, :]
    wd['shb2'] = p['sh_b2'][None, :]
    wd['wc1'] = wc1
    wd['bc1'] = bc1
    wd['wc2'] = wc2
    wd['bc2'] = bc2
    return _run_mlp(xg_sum, cnt, p, wd)


# speedup vs baseline: 24.2080x; 1.6252x over previous
import functools

import jax
import jax.numpy as jnp
import numpy as np
from jax import lax
from jax.experimental import pallas as pl
from jax.experimental.pallas import tpu as pltpu
from jax.experimental.pallas import tpu_sc as plsc

N = 10000
NP = 10240
NT_TILES = NP // 128
E = 320000
DF = 128
DE = 16
H = 128
NH = 8
HD = 16
NL = 4
NTASK = 5
NG = 64
EC = 1280
NCHUNK = E // EC
EHALF = E // 2
NCHUNK_H = EHALF // EC
ETILE = 512
NE_TILES = E // ETILE

_SC_PARAMS = pltpu.CompilerParams(needs_layout_passes=False)


def _eye(n):
    a = lax.broadcasted_iota(jnp.int32, (n, n), 0)
    b = lax.broadcasted_iota(jnp.int32, (n, n), 1)
    return (a == b).astype(jnp.float32)


_PREC = lax.Precision.DEFAULT


def _tr(x):
    return lax.dot_general(x, _eye(x.shape[0]), (((0,), (0,)), ((), ())),
                           preferred_element_type=jnp.float32, precision=_PREC)


def _dot(a, b):
    return lax.dot_general(a, b, (((1,), (0,)), ((), ())),
                           preferred_element_type=jnp.float32, precision=_PREC)


def _col_mask(rows, r):
    c = lax.broadcasted_iota(jnp.int32, (rows, 128), 1) + r * 128
    return c < N


def _prep_kernel(attr_ref, m1t_ref, call_ref, attrT_ref, aeT_ref, aemax_ref):
    e = pl.program_id(0)
    blk = attr_ref[...]
    t = lax.dot_general(blk, _eye(ETILE), (((0,), (0,)), ((), ())),
                        preferred_element_type=jnp.float32, precision=_PREC)
    attrT_ref[...] = t
    ae = _dot(m1t_ref[...], t) + call_ref[...][:, 0:1]
    aeT_ref[...] = ae
    m = jnp.max(ae.reshape(32, ETILE // 128, 128), axis=1)

    @pl.when(e == 0)
    def _():
        aemax_ref[...] = jnp.full((32, 128), -3e38, jnp.float32)

    aemax_ref[...] = jnp.maximum(aemax_ref[...], m)


def _run_prep(edge_attr, M1_T_all, c_all2d):
    return pl.pallas_call(
        _prep_kernel,
        grid=(NE_TILES,),
        in_specs=[
            pl.BlockSpec((ETILE, DE), lambda e: (e, 0)),
            pl.BlockSpec((32, DE), lambda e: (0, 0)),
            pl.BlockSpec((32, 128), lambda e: (0, 0)),
        ],
        out_specs=[
            pl.BlockSpec((DE, ETILE), lambda e: (0, e)),
            pl.BlockSpec((32, ETILE), lambda e: (0, e)),
            pl.BlockSpec((32, 128), lambda e: (0, 0)),
        ],
        out_shape=[
            jax.ShapeDtypeStruct((DE, E), jnp.float32),
            jax.ShapeDtypeStruct((32, E), jnp.float32),
            jax.ShapeDtypeStruct((32, 128), jnp.float32),
        ],
    )(edge_attr, M1_T_all, c_all2d)


def _sc_avg_build():
    mesh = plsc.VectorSubcoreMesh(core_axis_name="c", subcore_axis_name="s")

    @functools.partial(
        pl.kernel, mesh=mesh, compiler_params=_SC_PARAMS,
        out_type=(jax.ShapeDtypeStruct((32 * NP,), jnp.float32),
                  jax.ShapeDtypeStruct((2 * NP,), jnp.float32)),
        scratch_types=[
            pltpu.VMEM((NP,), jnp.float32),
            pltpu.VMEM((NP,), jnp.float32),
            pltpu.VMEM((EC,), jnp.int32),
            pltpu.VMEM((EC,), jnp.float32),
        ],
    )
    def k(attrT_hbm, dst_hbm, s_hbm, deg_hbm, acc_v, deg_v, dst_v, val_v):
        wid = lax.axis_index("s") * 2 + lax.axis_index("c")
        f = wid % 16
        h2 = wid // 16
        ebase = h2 * EHALF
        zero = jnp.zeros((16,), jnp.float32)
        ones = jnp.ones((16,), jnp.float32)

        def zinit(i, _):
            acc_v[pl.ds(i * 16, 16)] = zero
            deg_v[pl.ds(i * 16, 16)] = zero
            return 0

        lax.fori_loop(0, NP // 16, zinit, 0)

        def chunk(ci, _):
            base = ebase + ci * EC
            pltpu.sync_copy(dst_hbm.at[pl.ds(base, EC)], dst_v)
            pltpu.sync_copy(attrT_hbm.at[pl.ds(f * E + base, EC)], val_v)

            def inner(i, _):
                d16 = dst_v[pl.ds(i * 16, 16)]
                v16 = val_v[pl.ds(i * 16, 16)]
                plsc.addupdate_scatter(acc_v, [d16], v16)

                @pl.when(f == 0)
                def _():
                    plsc.addupdate_scatter(deg_v, [d16], ones)

                return 0

            lax.fori_loop(0, EC // 16, inner, 0, unroll=4)
            return 0

        lax.fori_loop(0, NCHUNK_H, chunk, 0)
        pltpu.sync_copy(acc_v, s_hbm.at[pl.ds((h2 * 16 + f) * NP, NP)])

        @pl.when(f == 0)
        def _():
            pltpu.sync_copy(deg_v, deg_hbm.at[pl.ds(h2 * NP, NP)])

    return k


def _sc_gat_build(layer):
    mesh = plsc.VectorSubcoreMesh(core_axis_name="c", subcore_axis_name="s")

    @functools.partial(
        pl.kernel, mesh=mesh, compiler_params=_SC_PARAMS,
        out_type=(jax.ShapeDtypeStruct((128 * NP,), jnp.float32),
                  jax.ShapeDtypeStruct((8 * NP,), jnp.float32)),
        scratch_types=[
            pltpu.VMEM((4 * NP,), jnp.float32),
            pltpu.VMEM((4 * NP,), jnp.float32),
            pltpu.VMEM((NP,), jnp.float32),
            pltpu.VMEM((NP,), jnp.float32),
            pltpu.VMEM((NP,), jnp.float32),
            pltpu.VMEM((2 * EC,), jnp.int32),
            pltpu.VMEM((2 * EC,), jnp.int32),
            pltpu.VMEM((2 * EC,), jnp.float32),
            pltpu.VMEM((128,), jnp.float32),
            pltpu.VMEM((128,), jnp.float32),
            pltpu.VMEM((128,), jnp.float32),
            pltpu.VMEM((128,), jnp.float32),
            pltpu.SemaphoreType.DMA,
            pltpu.SemaphoreType.DMA,
        ],
    )
    def k(xp_hbm, asrc_hbm, adst_hbm, ae_hbm, src_hbm, dst_hbm,
          cs_hbm, cd_hbm, cf_hbm, ce_hbm,
          raw_hbm, den_hbm,
          xp_v, acc_v, asrc_v, adst_v, den_v, src_v, dst_v, ae_v,
          cs_v, cd_v, cf_v, ce_v, sem0, sem1):
        wid = lax.axis_index("s") * 2 + lax.axis_index("c")
        h = wid // 4
        q = wid % 4
        f0 = wid * 4
        pltpu.sync_copy(xp_hbm.at[pl.ds(f0 * NP, 4 * NP)], xp_v)
        pltpu.sync_copy(asrc_hbm.at[pl.ds(h * NP, NP)], asrc_v)
        pltpu.sync_copy(adst_hbm.at[pl.ds(h * NP, NP)], adst_v)
        pltpu.sync_copy(cs_hbm.at[pl.ds(h * 128, 128)], cs_v)
        pltpu.sync_copy(cd_hbm.at[pl.ds(h * 128, 128)], cd_v)
        pltpu.sync_copy(cf_hbm.at[pl.ds(h * 128, 128)], cf_v)
        pltpu.sync_copy(ce_hbm.at[pl.ds(h * 128, 128)], ce_v)

        def vmax(v):
            def mstep(i, m):
                return jnp.maximum(m, v[pl.ds(i * 16, 16)])
            m = lax.fori_loop(0, 8, mstep, jnp.full((16,), -3e38, jnp.float32))
            return jnp.max(m)

        c = vmax(cs_v) + vmax(cd_v) + jnp.maximum(vmax(cf_v), vmax(ce_v))
        cvec = jnp.broadcast_to(c, (16,))

        ae_base = (8 * layer + h) * E

        def copies(ci, slot, sem):
            base = ci * EC
            off = slot * EC
            pltpu.make_async_copy(src_hbm.at[pl.ds(base, EC)],
                                  src_v.at[pl.ds(off, EC)], sem).start()
            pltpu.make_async_copy(dst_hbm.at[pl.ds(base, EC)],
                                  dst_v.at[pl.ds(off, EC)], sem).start()
            pltpu.make_async_copy(ae_hbm.at[pl.ds(ae_base + base, EC)],
                                  ae_v.at[pl.ds(off, EC)], sem).start()

        def waits(ci, slot, sem):
            base = ci * EC
            off = slot * EC
            pltpu.make_async_copy(src_hbm.at[pl.ds(base, EC)],
                                  src_v.at[pl.ds(off, EC)], sem).wait()
            pltpu.make_async_copy(dst_hbm.at[pl.ds(base, EC)],
                                  dst_v.at[pl.ds(off, EC)], sem).wait()
            pltpu.make_async_copy(ae_hbm.at[pl.ds(ae_base + base, EC)],
                                  ae_v.at[pl.ds(off, EC)], sem).wait()

        copies(0, 0, sem0)

        zero = jnp.zeros((16,), jnp.float32)

        def zinit(i, _):
            acc_v[pl.ds(i * 16, 16)] = zero
            return 0

        lax.fori_loop(0, 4 * NP // 16, zinit, 0)

        def zden(i, _):
            den_v[pl.ds(i * 16, 16)] = zero
            return 0

        lax.fori_loop(0, NP // 16, zden, 0)

        def chunk(ci, _):
            slot = ci % 2
            off = slot * EC

            @pl.when(ci + 1 < NCHUNK)
            def _():
                @pl.when(slot == 0)
                def _():
                    copies(ci + 1, 1, sem1)

                @pl.when(slot == 1)
                def _():
                    copies(ci + 1, 0, sem0)

            @pl.when(slot == 0)
            def _():
                waits(ci, 0, sem0)

            @pl.when(slot == 1)
            def _():
                waits(ci, 1, sem1)

            def exb(i, _):
                b = off + i * 16
                s16 = src_v[pl.ds(b, 16)]
                d16 = dst_v[pl.ds(b, 16)]
                a = (plsc.load_gather(asrc_v, [s16])
                     + plsc.load_gather(adst_v, [d16])
                     + ae_v[pl.ds(b, 16)])
                a = jnp.where(a > 0, a, 0.2 * a)
                ex = jnp.exp(a - cvec)

                @pl.when(q == 0)
                def _():
                    plsc.addupdate_scatter(den_v, [d16], ex)

                v0 = plsc.load_gather(xp_v, [s16])
                plsc.addupdate_scatter(acc_v, [d16], v0 * ex)
                s1 = s16 + NP
                d1 = d16 + NP
                v1 = plsc.load_gather(xp_v, [s1])
                plsc.addupdate_scatter(acc_v, [d1], v1 * ex)
                s2 = s1 + NP
                d2 = d1 + NP
                v2 = plsc.load_gather(xp_v, [s2])
                plsc.addupdate_scatter(acc_v, [d2], v2 * ex)
                s3 = s2 + NP
                d3 = d2 + NP
                v3 = plsc.load_gather(xp_v, [s3])
                plsc.addupdate_scatter(acc_v, [d3], v3 * ex)
                return 0

            lax.fori_loop(0, EC // 16, exb, 0, unroll=8)
            return 0

        lax.fori_loop(0, NCHUNK, chunk, 0)
        pltpu.sync_copy(acc_v, raw_hbm.at[pl.ds(f0 * NP, 4 * NP)])

        @pl.when(q == 0)
        def _():
            pltpu.sync_copy(den_v, den_hbm.at[pl.ds(h * NP, NP)])

    return k


def _a_phase(r, hT, avgT, hd8, wT, asT, adT, m1T, ccol,
             xp_ref, asrc_ref, adst_ref, aself_ref, cs_ref, cd_ref, cf_ref):
    xp = _dot(wT, hT)
    asrc = _dot(asT, hT)
    adst = _dot(adT, hT)
    aself = _dot(m1T, avgT) + ccol * hd8
    xp_ref[...] = xp
    asrc_ref[...] = asrc
    adst_ref[...] = adst
    aself_ref[...] = aself

    @pl.when(r == 0)
    def _():
        neg = jnp.full((8, 128), -3e38, jnp.float32)
        cs_ref[...] = neg
        cd_ref[...] = neg
        cf_ref[...] = neg

    cs_ref[...] = jnp.maximum(cs_ref[...], asrc)
    cd_ref[...] = jnp.maximum(cd_ref[...], adst)
    cf_ref[...] = jnp.maximum(cf_ref[...], aself)


_A_OUT_SHAPES = [
    jax.ShapeDtypeStruct((H, NP), jnp.float32),
    jax.ShapeDtypeStruct((8, NP), jnp.float32),
    jax.ShapeDtypeStruct((8, NP), jnp.float32),
    jax.ShapeDtypeStruct((8, NP), jnp.float32),
    jax.ShapeDtypeStruct((8, 128), jnp.float32),
    jax.ShapeDtypeStruct((8, 128), jnp.float32),
    jax.ShapeDtypeStruct((8, 128), jnp.float32),
]

_A_OUT_SPECS = [
    pl.BlockSpec((H, 128), lambda r: (0, r)),
    pl.BlockSpec((8, 128), lambda r: (0, r)),
    pl.BlockSpec((8, 128), lambda r: (0, r)),
    pl.BlockSpec((8, 128), lambda r: (0, r)),
    pl.BlockSpec((8, 128), lambda r: (0, 0)),
    pl.BlockSpec((8, 128), lambda r: (0, 0)),
    pl.BlockSpec((8, 128), lambda r: (0, 0)),
]

_W_SPEC = pl.BlockSpec((H, H), lambda r: (0, 0))
_W8_SPEC = pl.BlockSpec((8, H), lambda r: (0, 0))
_C_SPEC = pl.BlockSpec((8, 128), lambda r: (0, 0))
_TILE_SPEC = pl.BlockSpec((H, 128), lambda r: (0, r))
_ROW8_SPEC = pl.BlockSpec((8, 128), lambda r: (0, r))


def _a0_kernel(x_ref, s_ref, deg_ref, watom_ref, batom_ref,
               wT_ref, asT_ref, adT_ref, m1T_ref, ccol_ref,
               hT_ref, avgT_ref, hd8_ref,
               xp_ref, asrc_ref, adst_ref, aself_ref, cs_ref, cd_ref, cf_ref):
    r = pl.program_id(0)
    h0 = _dot(x_ref[...], watom_ref[...]) + batom_ref[...]
    hT = _tr(h0)
    hT = jnp.where(_col_mask(H, r), hT, 0.0)
    hT_ref[...] = hT
    s = s_ref[...]
    d2 = deg_ref[...]
    d = d2[0:1] + d2[1:2]
    avgT = (s[0] + s[1]) / jnp.maximum(d, 1.0)
    hd8 = jnp.broadcast_to((d > 0).astype(jnp.float32), (8, 128))
    avgT_ref[...] = avgT
    hd8_ref[...] = hd8
    _a_phase(r, hT, avgT, hd8, wT_ref[...], asT_ref[...], adT_ref[...],
             m1T_ref[...], ccol_ref[...][:, 0:1],
             xp_ref, asrc_ref, adst_ref, aself_ref, cs_ref, cd_ref, cf_ref)


def _run_a0(x_p, S3, degp, p, wd):
    return pl.pallas_call(
        _a0_kernel,
        grid=(NT_TILES,),
        in_specs=[
            pl.BlockSpec((128, DF), lambda r: (r, 0)),
            pl.BlockSpec((2, 16, 128), lambda r: (0, 0, r)),
            pl.BlockSpec((2, 128), lambda r: (0, r)),
            _W_SPEC,
            _W_SPEC,
            _W_SPEC, _W8_SPEC, _W8_SPEC,
            pl.BlockSpec((8, DE), lambda r: (0, 0)),
            _C_SPEC,
        ],
        out_specs=[_TILE_SPEC, pl.BlockSpec((16, 128), lambda r: (0, r)),
                   _ROW8_SPEC] + _A_OUT_SPECS,
        out_shape=[
            jax.ShapeDtypeStruct((H, NP), jnp.float32),
            jax.ShapeDtypeStruct((16, NP), jnp.float32),
            jax.ShapeDtypeStruct((8, NP), jnp.float32),
        ] + _A_OUT_SHAPES,
    )(x_p, S3, degp, p['W_atom'], wd['batom2d'],
      wd['wT'][0], wd['asT'][0], wd['adT'][0], wd['m1T'][0], wd['ccol'][0])


def _b1_kernel(raw_ref, den_ref, xp_ref, asrc_ref, adst_ref, aself_ref,
               cs_ref, cd_ref, cf_ref, ce_ref, bias_ref,
               o_ref, sacc_ref, ssacc_ref):
    r = pl.program_id(0)
    ch = (jnp.max(cs_ref[...], axis=1, keepdims=True)
          + jnp.max(cd_ref[...], axis=1, keepdims=True)
          + jnp.maximum(jnp.max(cf_ref[...], axis=1, keepdims=True),
                        jnp.max(ce_ref[...], axis=1, keepdims=True)))
    al = asrc_ref[...] + adst_ref[...] + aself_ref[...]
    al = jnp.where(al > 0, al, 0.2 * al)
    exs = jnp.exp(al - ch)
    dent = den_ref[...] + exs
    e128 = jnp.reshape(jnp.broadcast_to(exs[:, None, :], (8, 16, 128)), (128, 128))
    d128 = jnp.reshape(jnp.broadcast_to(dent[:, None, :], (8, 16, 128)), (128, 128))
    o = (raw_ref[...] + e128 * xp_ref[...]) / d128 + bias_ref[...]
    om = jnp.where(_col_mask(H, r), o, 0.0)
    o_ref[...] = om

    @pl.when(r == 0)
    def _():
        z = jnp.zeros((H, 128), jnp.float32)
        sacc_ref[...] = z
        ssacc_ref[...] = z

    sacc_ref[...] = sacc_ref[...] + om
    ssacc_ref[...] = ssacc_ref[...] + om * om


def _run_b1(rawT, denT, xpT, asrcT, adstT, aselfT, cs, cd, cf, ce, bias2d):
    return pl.pallas_call(
        _b1_kernel,
        grid=(NT_TILES,),
        in_specs=[_TILE_SPEC, _ROW8_SPEC, _TILE_SPEC, _ROW8_SPEC, _ROW8_SPEC,
                  _ROW8_SPEC, _C_SPEC, _C_SPEC, _C_SPEC, _C_SPEC, _W_SPEC],
        out_specs=[_TILE_SPEC, pl.BlockSpec((H, 128), lambda r: (0, 0)),
                   pl.BlockSpec((H, 128), lambda r: (0, 0))],
        out_shape=[jax.ShapeDtypeStruct((H, NP), jnp.float32),
                   jax.ShapeDtypeStruct((H, 128), jnp.float32),
                   jax.ShapeDtypeStruct((H, 128), jnp.float32)],
    )(rawT, denT, xpT, asrcT, adstT, aselfT, cs, cd, cf, ce, bias2d)


def _make_b2_kernel(layer, last):
    def body(o_ref, sacc_ref, ssacc_ref, g_ref, b_ref, hres_ref,
             avgT_ref, hd8_ref, wT_ref, asT_ref, adT_ref, m1T_ref, ccol_ref,
             hT_ref, *a_refs):
        r = pl.program_id(0)
        srow = jnp.sum(sacc_ref[...], axis=1, keepdims=True)
        ssrow = jnp.sum(ssacc_ref[...], axis=1, keepdims=True)
        mu = srow / float(N)
        var = ssrow / float(N) - mu * mu
        inv = lax.rsqrt(var + 1e-5)
        hn = (o_ref[...] - mu) * inv * g_ref[...] + b_ref[...]
        hn = jnp.maximum(hn, 0.0)
        if layer > 0:
            hn = hn + hres_ref[...]
        hn = jnp.where(_col_mask(H, r), hn, 0.0)
        hT_ref[...] = hn
        if not last:
            _a_phase(r, hn, avgT_ref[...], hd8_ref[...], wT_ref[...],
                     asT_ref[...], adT_ref[...], m1T_ref[...],
                     ccol_ref[...][:, 0:1], *a_refs)
    return body


def _run_b2(layer, oT, sacc, ssacc, g2d, b2d, hresT, avgT, hd8T, wd):
    last = layer == NL - 1
    nxt = layer + 1
    out_specs = [_TILE_SPEC]
    out_shape = [jax.ShapeDtypeStruct((H, NP), jnp.float32)]
    if not last:
        out_specs += _A_OUT_SPECS
        out_shape += _A_OUT_SHAPES
    return pl.pallas_call(
        _make_b2_kernel(layer, last),
        grid=(NT_TILES,),
        in_specs=[_TILE_SPEC,
                  pl.BlockSpec((H, 128), lambda r: (0, 0)),
                  pl.BlockSpec((H, 128), lambda r: (0, 0)),
                  _W_SPEC, _W_SPEC, _TILE_SPEC,
                  pl.BlockSpec((16, 128), lambda r: (0, r)),
                  _ROW8_SPEC,
                  _W_SPEC, _W8_SPEC, _W8_SPEC,
                  pl.BlockSpec((8, DE), lambda r: (0, 0)),
                  _C_SPEC],
        out_specs=out_specs,
        out_shape=out_shape,
    )(oT, sacc, ssacc, g2d, b2d, hresT, avgT, hd8T,
      wd['wT'][nxt % NL], wd['asT'][nxt % NL], wd['adT'][nxt % NL],
      wd['m1T'][nxt % NL], wd['ccol'][nxt % NL])


def _qkv_kernel(hT_ref, wqT_ref, wkT_ref, wvT_ref, bq_ref, bk_ref, bv_ref,
                q_ref, k_ref, v_ref):
    hnat = _tr(hT_ref[...])
    q_ref[...] = _dot(hnat, wqT_ref[...]) + bq_ref[...]
    k_ref[...] = _dot(hnat, wkT_ref[...]) + bk_ref[...]
    v_ref[...] = _dot(hnat, wvT_ref[...]) + bv_ref[...]


def _run_qkv(hT, wqT, wkT, wvT, bq2d, bk2d, bv2d):
    spec_nat = pl.BlockSpec((128, H), lambda r: (r, 0))
    return pl.pallas_call(
        _qkv_kernel,
        grid=(NT_TILES,),
        in_specs=[_TILE_SPEC] + [_W_SPEC] * 6,
        out_specs=[spec_nat] * 3,
        out_shape=[jax.ShapeDtypeStruct((NP, H), jnp.float32)] * 3,
    )(hT, wqT, wkT, wvT, bq2d, bk2d, bv2d)


def _mha_kernel(q_ref, k_ref, v_ref, brow_ref, btile_ref, woT_ref, bo_ref,
                xg_ref, cnt_ref):
    r = pl.program_id(0)
    btile = btile_ref[...]
    brT = lax.dot_general(_eye(128), btile, (((1,), (1,)), ((), ())),
                          preferred_element_type=jnp.float32, precision=_PREC)
    brow = brow_ref[...]
    mask = brT == brow
    q = q_ref[...]
    outs = []
    for hh in range(NH):
        sl = slice(hh * HD, (hh + 1) * HD)
        sc = lax.dot_general(q[:, sl], k_ref[:, sl],
                             (((1,), (1,)), ((), ())),
                             preferred_element_type=jnp.float32,
                             precision=_PREC) * 0.25
        sc = jnp.where(mask, sc, -1e30)
        mx = jnp.max(sc, axis=1, keepdims=True)
        pexp = jnp.exp(sc - mx)
        den = jnp.sum(pexp, axis=1, keepdims=True)
        at = pexp / den
        outs.append(_dot(at, v_ref[:, sl]))
    o = jnp.concatenate(outs, axis=1)
    att = _dot(o, woT_ref[...]) + bo_ref[...]
    giota = lax.broadcasted_iota(jnp.int32, (NG, 128), 0).astype(jnp.float32)
    oneh = (giota == btile).astype(jnp.float32)
    xg = lax.dot_general(oneh, att, (((1,), (0,)), ((), ())),
                         preferred_element_type=jnp.float32,
                         precision=lax.Precision.HIGHEST)
    cnt = lax.dot_general(oneh, jnp.ones((128, 128), jnp.float32),
                          (((1,), (0,)), ((), ())),
                          preferred_element_type=jnp.float32,
                          precision=lax.Precision.HIGHEST)

    @pl.when(r == 0)
    def _():
        z = jnp.zeros((NG, 128), jnp.float32)
        xg_ref[...] = z
        cnt_ref[...] = z

    xg_ref[...] = xg_ref[...] + xg
    cnt_ref[...] = cnt_ref[...] + cnt


def _run_mha(q, k, v, brow, woT, bo2d):
    return pl.pallas_call(
        _mha_kernel,
        grid=(NT_TILES,),
        in_specs=[
            pl.BlockSpec((128, H), lambda r: (r, 0)),
            pl.BlockSpec((NP, H), lambda r: (0, 0)),
            pl.BlockSpec((NP, H), lambda r: (0, 0)),
            pl.BlockSpec((1, NP), lambda r: (0, 0)),
            pl.BlockSpec((1, 128), lambda r: (0, r)),
            _W_SPEC, _W_SPEC,
        ],
        out_specs=[pl.BlockSpec((NG, 128), lambda r: (0, 0)),
                   pl.BlockSpec((NG, 128), lambda r: (0, 0))],
        out_shape=[jax.ShapeDtypeStruct((NG, 128), jnp.float32),
                   jax.ShapeDtypeStruct((NG, 128), jnp.float32)],
    )(q, k, v, brow, brow, woT, bo2d)


def _mlp_kernel(xg_ref, cnt_ref, w1_ref, b1_ref, w2_ref, b2_ref,
                wc1_ref, bc1_ref, wc2_ref, bc2_ref, out_ref):
    xg = xg_ref[...] / cnt_ref[...]
    s = jnp.maximum(_dot(xg, w1_ref[...]) + b1_ref[...], 0.0)
    s = jnp.maximum(_dot(s, w2_ref[...]) + b2_ref[...], 0.0)
    hh = jnp.maximum(_dot(s, wc1_ref[...]) + bc1_ref[...], 0.0)
    out_ref[...] = _dot(hh, wc2_ref[...]) + bc2_ref[...]


def _run_mlp(xg_sum, cnt, p, wd):
    return pl.pallas_call(
        _mlp_kernel,
        out_shape=jax.ShapeDtypeStruct((NG, NTASK), jnp.float32),
    )(xg_sum, cnt, p['sh_W1'], wd['shb1'], p['sh_W2'], wd['shb2'],
      wd['wc1'], wd['bc1'], wd['wc2'], wd['bc2'])


def kernel(x, edge_attr, params, edge_index, batch):
    p = params
    src = edge_index[0].astype(jnp.int32)
    dst = edge_index[1].astype(jnp.int32)

    asT, adT, m1T, ccol, wT = [], [], [], [], []
    M1_rows, c_list = [], []
    for i in range(NL):
        W = p[f'gat{i}_W']
        a_s = (W.reshape(H, NH, HD) * p[f'gat{i}_att_src'][0][None]).sum(-1)
        a_d = (W.reshape(H, NH, HD) * p[f'gat{i}_att_dst'][0][None]).sum(-1)
        Ep = (p[f'gat{i}_W_e'].reshape(H, NH, HD)
              * p[f'gat{i}_att_edge'][0][None]).sum(-1)
        M1 = p['W_edge_enc'] @ Ep
        cv = p['b_edge_enc'] @ Ep
        asT.append(a_s.T)
        adT.append(a_d.T)
        m1T.append(M1.T)
        ccol.append(jnp.broadcast_to(cv[:, None], (8, 128)))
        wT.append(W.T)
        M1_rows.append(M1.T)
        c_list.append(cv)
    wd = {
        'wT': wT, 'asT': asT, 'adT': adT, 'm1T': m1T, 'ccol': ccol,
        'batom2d': jnp.broadcast_to(p['b_atom'][None, :], (H, H)),
    }
    M1_T_all = jnp.concatenate(M1_rows, axis=0)
    c_all2d = jnp.broadcast_to(jnp.concatenate(c_list)[:, None], (32, 128))

    x_p = jnp.pad(x, ((0, NP - N), (0, 0)))
    batch_p = jnp.pad(batch.astype(jnp.int32), (0, NP - N),
                      constant_values=127)
    brow = batch_p.astype(jnp.float32).reshape(1, NP)

    attrT, aeT, aemax = _run_prep(edge_attr, M1_T_all, c_all2d)

    S_f, deg_f = _sc_avg_build()(attrT.reshape(-1), dst)
    S3 = S_f.reshape(2, 16, NP)
    degp = deg_f.reshape(2, NP)

    (hT, avgT, hd8T, xpT, asrcT, adstT, aselfT, cs, cd, cf) = _run_a0(
        x_p, S3, degp, p, wd)

    ae_flat = aeT.reshape(-1)
    for i in range(NL):
        ce = aemax[8 * i:8 * (i + 1)]
        raw_f, den_f = _sc_gat_build(i)(
            xpT.reshape(-1), asrcT.reshape(-1), adstT.reshape(-1), ae_flat,
            src, dst, cs.reshape(-1), cd.reshape(-1), cf.reshape(-1),
            ce.reshape(-1))
        rawT = raw_f.reshape(H, NP)
        denT = den_f.reshape(8, NP)
        bias2d = jnp.broadcast_to(p[f'gat{i}_bias'][:, None], (H, H))
        oT, sacc, ssacc = _run_b1(rawT, denT, xpT, asrcT, adstT, aselfT,
                                  cs, cd, cf, ce, bias2d)
        g2d = jnp.broadcast_to(p[f'bn{i}_g'][:, None], (H, H))
        b2d = jnp.broadcast_to(p[f'bn{i}_b'][:, None], (H, H))
        res = _run_b2(i, oT, sacc, ssacc, g2d, b2d, hT, avgT, hd8T, wd)
        if i < NL - 1:
            hT, xpT, asrcT, adstT, aselfT, cs, cd, cf = res
        else:
            hT = res[0]

    Wq, Wk, Wv = jnp.split(p['mha_in_w'], 3, axis=0)
    bq, bk, bv = jnp.split(p['mha_in_b'], 3)
    q, k, v = _run_qkv(hT, Wq.T, Wk.T, Wv.T,
                       jnp.broadcast_to(bq[None, :], (H, H)),
                       jnp.broadcast_to(bk[None, :], (H, H)),
                       jnp.broadcast_to(bv[None, :], (H, H)))
    xg_sum, cnt = _run_mha(q, k, v, brow, p['mha_out_w'].T,
                           jnp.broadcast_to(p['mha_out_b'][None, :], (H, H)))

    wc1 = jnp.concatenate([p[f'head{t}_W1'] for t in range(NTASK)], axis=1)
    bc1 = jnp.concatenate([p[f'head{t}_b1'] for t in range(NTASK)])[None, :]
    blocks = []
    for t in range(NTASK):
        col = jnp.zeros((H // 4, NTASK), jnp.float32)
        col = col.at[:, t].set(p[f'head{t}_W2'][:, 0])
        blocks.append(col)
    wc2 = jnp.concatenate(blocks, axis=0)
    bc2 = jnp.concatenate([p[f'head{t}_b2'] for t in range(NTASK)])[None, :]
    wd['shb1'] = p['sh_b1'][None, :]
    wd['shb2'] = p['sh_b2'][None, :]
    wd['wc1'] = wc1
    wd['bc1'] = bc1
    wd['wc2'] = wc2
    wd['bc2'] = bc2
    return _run_mlp(xg_sum, cnt, p, wd)
